# Initial kernel scaffold; baseline (speedup 1.0000x reference)
import jax
import jax.numpy as jnp
from jax.experimental import pallas as pl


def kernel(features, edge_index, edge_weight, W1, b1, W2, b2):
    raise NotImplementedError("write your pallas kernel here")


# speedup vs baseline: 3.1196x; 3.1196x over previous
import functools

import jax
import jax.numpy as jnp
from jax import lax
from jax.experimental import pallas as pl
from jax.experimental.pallas import tpu as pltpu
from jax.experimental.pallas import tpu_sc as plsc

NC = 2
NS = 16
NW = NC * NS
LN = 16


def _deg_kernel(col, w, zeros, idgrid):
    E = col.shape[0]
    EW = E // NW
    steps = EW // LN
    NP = 10240
    ept = NP // NS
    mesh = plsc.VectorSubcoreMesh(core_axis_name="c", subcore_axis_name="s", num_cores=NC, num_subcores=NS)

    @functools.partial(
        pl.kernel, mesh=mesh,
        compiler_params=pltpu.CompilerParams(needs_layout_passes=False, use_tc_tiling_on_sc=False),
        out_type=jax.ShapeDtypeStruct((NC, NP), jnp.float32),
        scratch_types=[
            pltpu.VMEM((EW,), jnp.int32),
            pltpu.VMEM((EW,), jnp.float32),
            pltpu.VMEM((NP,), jnp.float32),
            pltpu.VMEM((NP // 128, 128), jnp.int32),
            pltpu.VMEM_SHARED((NP,), jnp.float32),
        ],
    )
    def k(col_h, w_h, z_h, idg_h, out_h, col_v, w_v, dloc, idg_v, dacc):
        cid = lax.axis_index("c")
        sid = lax.axis_index("s")
        wid = sid * NC + cid
        base = wid * EW
        pltpu.sync_copy(z_h.at[pl.ds(0, ept)], dacc.at[pl.ds(sid * ept, ept)])
        pltpu.sync_copy(z_h, dloc)
        pltpu.sync_copy(idg_h, idg_v)
        pltpu.sync_copy(col_h.at[pl.ds(base, EW)], col_v)
        pltpu.sync_copy(w_h.at[pl.ds(base, EW)], w_v)

        def step(i, carry):
            ib = i * LN
            c = col_v[pl.ds(ib, LN)]
            wv = w_v[pl.ds(ib, LN)]
            plsc.addupdate_scatter(dloc, [c], wv)
            return carry

        lax.fori_loop(0, steps, step, 0)
        plsc.subcore_barrier()
        for j in range(NP // 128):
            pltpu.sync_copy(dloc.at[pl.ds(j * 128, 128)],
                            dacc.at[idg_v.at[j]], add=True)
        plsc.subcore_barrier()
        pltpu.sync_copy(dacc.at[pl.ds(sid * ept, ept)],
                        out_h.at[cid, pl.ds(sid * ept, ept)])

    return k(col, w, zeros, idgrid)


def _dis_kernel(degp):
    def body(dp_ref, o_ref):
        deg = dp_ref[0] + dp_ref[1] + 1.0
        o_ref[...] = jnp.where(deg > 0,
                               lax.rsqrt(jnp.maximum(deg, 1e-12)), 0.0)

    return pl.pallas_call(
        body,
        out_shape=jax.ShapeDtypeStruct(degp.shape[1:], jnp.float32),
    )(degp)


def _matmul(x, wT, br=1000):
    n, kdim = x.shape
    m = wT.shape[1]

    def body(x_ref, w_ref, o_ref):
        o_ref[...] = jnp.dot(x_ref[...], w_ref[...],
                             preferred_element_type=jnp.float32)

    return pl.pallas_call(
        body,
        grid=(n // br,),
        in_specs=[pl.BlockSpec((br, kdim), lambda i: (i, 0)),
                  pl.BlockSpec((kdim, m), lambda i: (0, 0))],
        out_specs=pl.BlockSpec((br, m), lambda i: (i, 0)),
        out_shape=jax.ShapeDtypeStruct((n, m), jnp.float32),
    )(x, wT)


def _prop(table4, row, col, wnorm, dis, zeros, gpc):
    nt, dq = table4.shape
    n = nt // (NC * gpc)
    e = row.shape[0]
    ew = e // NS
    steps = ew // LN
    rpt = n // NS
    compute_norm = dis is not None
    mesh = plsc.VectorSubcoreMesh(core_axis_name="c", subcore_axis_name="s", num_cores=NC, num_subcores=NS)

    out_type = jax.ShapeDtypeStruct((NC * gpc, n, dq), jnp.float32)
    if compute_norm:
        out_type = (out_type, jax.ShapeDtypeStruct((e,), jnp.float32))
    scratch = [
        pltpu.VMEM((ew,), jnp.int32),
        pltpu.VMEM((ew,), jnp.int32),
        pltpu.VMEM((ew,), jnp.float32),
        pltpu.VMEM((LN, dq), jnp.float32),
        pltpu.VMEM_SHARED((n, dq), jnp.float32),
        pltpu.SemaphoreType.DMA,
    ]
    if compute_norm:
        scratch.insert(3, pltpu.VMEM((dis.shape[0],), jnp.float32))

    @functools.partial(
        pl.kernel, mesh=mesh,
        compiler_params=pltpu.CompilerParams(needs_layout_passes=False, use_tc_tiling_on_sc=False),
        out_type=out_type, scratch_types=scratch,
    )
    def k(*refs):
        if compute_norm:
            (tab_h, row_h, col_h, w_h, dis_h, z_h, part_h, nrm_h,
             row_v, col_v, nrm_v, dis_v, rv, acc, sem) = refs
        else:
            (tab_h, row_h, col_h, nrm_h_in, z_h, part_h,
             row_v, col_v, nrm_v, rv, acc, sem) = refs
        cid = lax.axis_index("c")
        sid = lax.axis_index("s")
        base = sid * ew
        zr = zeros.shape[0]
        pltpu.sync_copy(row_h.at[pl.ds(base, ew)], row_v)
        pltpu.sync_copy(col_h.at[pl.ds(base, ew)], col_v)
        if compute_norm:
            pltpu.sync_copy(w_h.at[pl.ds(base, ew)], nrm_v)
            pltpu.sync_copy(dis_h, dis_v)

            def nstep(i, carry):
                ib = i * LN
                r = row_v[pl.ds(ib, LN)]
                c = col_v[pl.ds(ib, LN)]
                wv = nrm_v[pl.ds(ib, LN)]
                dr = plsc.load_gather(dis_v, [r])
                dc = plsc.load_gather(dis_v, [c])
                nrm_v[pl.ds(ib, LN)] = dr * wv * dc
                return carry

            lax.fori_loop(0, steps, nstep, 0)
        else:
            pltpu.sync_copy(nrm_h_in.at[pl.ds(base, ew)], nrm_v)

        for g in range(gpc):
            grp = cid * gpc + g
            roff = grp * n
            for j in range(rpt // zr):
                pltpu.sync_copy(z_h, acc.at[pl.ds(sid * rpt + j * zr, zr)])
            plsc.subcore_barrier()

            def step(i, carry):
                ib = i * LN
                r = row_v[pl.ds(ib, LN)]
                c = col_v[pl.ds(ib, LN)]
                nv = nrm_v[pl.ds(ib, LN)]
                pltpu.async_copy(tab_h.at[r + roff], rv, sem).wait()
                for lane in range(LN):
                    s = nv[lane]
                    for jj in range(dq // LN):
                        rv[lane, pl.ds(jj * LN, LN)] = rv[lane, pl.ds(jj * LN, LN)] * s
                pltpu.sync_copy(rv, acc.at[c], add=True)
                return carry

            lax.fori_loop(0, steps, step, 0)
            plsc.subcore_barrier()
            pltpu.sync_copy(acc.at[pl.ds(sid * rpt, rpt)],
                            part_h.at[grp, pl.ds(sid * rpt, rpt)])

        if compute_norm:
            @pl.when(cid == 0)
            def _():
                pltpu.sync_copy(nrm_v, nrm_h.at[pl.ds(base, ew)])

    if compute_norm:
        return k(table4, row, col, wnorm, dis, zeros)
    return k(table4, row, col, wnorm, zeros)


def _tc2(part1, dis2d, xw1, b1row, w2T, br=1000):
    n, d = xw1.shape
    m = w2T.shape[1]

    ng = part1.shape[0]
    dh = d // ng

    def body(p_ref, d_ref, x_ref, b_ref, w_ref, o_ref):
        d2 = d_ref[...] * d_ref[...]
        agg = jnp.concatenate([p_ref[g] for g in range(ng)], axis=1)
        h = agg + d2 * x_ref[...] + b_ref[...]
        h = jnp.maximum(h, 0.0)
        o_ref[...] = jnp.dot(h, w_ref[...], preferred_element_type=jnp.float32)

    return pl.pallas_call(
        body,
        grid=(n // br,),
        in_specs=[pl.BlockSpec((ng, br, dh), lambda i: (0, i, 0)),
                  pl.BlockSpec((br, 1), lambda i: (i, 0)),
                  pl.BlockSpec((br, d), lambda i: (i, 0)),
                  pl.BlockSpec((1, d), lambda i: (0, 0)),
                  pl.BlockSpec((d, m), lambda i: (0, 0))],
        out_specs=pl.BlockSpec((br, m), lambda i: (i, 0)),
        out_shape=jax.ShapeDtypeStruct((n, m), jnp.float32),
    )(part1, dis2d, xw1, b1row, w2T)


def _tc3(part2, dis2d, h2, b2row, br=1000):
    n, m = h2.shape

    ng = part2.shape[0]
    mh = m // ng

    def body(p_ref, d_ref, h_ref, b_ref, o_ref):
        d2 = d_ref[...] * d_ref[...]
        agg = jnp.concatenate([p_ref[g] for g in range(ng)], axis=1)
        logit = agg + d2 * h_ref[...] + b_ref[...]
        mx = jnp.max(logit, axis=1, keepdims=True)
        lse = jnp.log(jnp.sum(jnp.exp(logit - mx), axis=1, keepdims=True)) + mx
        o_ref[...] = logit - lse

    return pl.pallas_call(
        body,
        grid=(n // br,),
        in_specs=[pl.BlockSpec((ng, br, mh), lambda i: (0, i, 0)),
                  pl.BlockSpec((br, 1), lambda i: (i, 0)),
                  pl.BlockSpec((br, m), lambda i: (i, 0)),
                  pl.BlockSpec((1, m), lambda i: (0, 0))],
        out_specs=pl.BlockSpec((br, m), lambda i: (i, 0)),
        out_shape=jax.ShapeDtypeStruct((n, m), jnp.float32),
    )(part2, dis2d, h2, b2row)


def kernel(features, edge_index, edge_weight, W1, b1, W2, b2):
    n, _ = features.shape
    row = edge_index[0]
    col = edge_index[1]

    z10240 = jnp.zeros((10240,), jnp.float32)
    idgrid = jnp.arange(10240, dtype=jnp.int32).reshape(80, 128)
    degp = _deg_kernel(col, edge_weight, z10240, idgrid)
    disg = _dis_kernel(degp.reshape(NC, 80, 128))
    dis_flat = disg.reshape(-1)
    dis2d = dis_flat[:n].reshape(n, 1)

    gpc = 2
    xw1 = _matmul(features, W1.T)
    d1 = xw1.shape[1] // (NC * gpc)
    tab1 = jnp.concatenate(
        [xw1[:, g * d1:(g + 1) * d1] for g in range(NC * gpc)], axis=0)
    z1 = jnp.zeros((125, d1), jnp.float32)
    part1, norm = _prop(tab1, row, col, edge_weight, dis_flat, z1, gpc)
    h2 = _tc2(part1, dis2d, xw1, b1.reshape(1, -1), W2.T)
    d2w = h2.shape[1] // (NC * gpc)
    tab2 = jnp.concatenate(
        [h2[:, g * d2w:(g + 1) * d2w] for g in range(NC * gpc)], axis=0)
    z2 = jnp.zeros((125, d2w), jnp.float32)
    part2 = _prop(tab2, row, col, norm, None, z2, gpc)
    return _tc3(part2, dis2d, h2, b2.reshape(1, -1))


# speedup vs baseline: 20.5356x; 6.5829x over previous
import functools

import jax
import jax.numpy as jnp
from jax import lax
from jax.experimental import pallas as pl
from jax.experimental.pallas import tpu as pltpu
from jax.experimental.pallas import tpu_sc as plsc

NC = 2
NS = 16
NW = NC * NS
LN = 16
GB = 80


def _deg_kernel(col, w, zeros, idgrid):
    E = col.shape[0]
    EW = E // NW
    steps = EW // LN
    NP = 10240
    ept = NP // NS
    mesh = plsc.VectorSubcoreMesh(core_axis_name="c", subcore_axis_name="s",
                                  num_cores=NC, num_subcores=NS)

    @functools.partial(
        pl.kernel, mesh=mesh,
        compiler_params=pltpu.CompilerParams(
            needs_layout_passes=False, use_tc_tiling_on_sc=False),
        out_type=jax.ShapeDtypeStruct((NC, NP), jnp.float32),
        scratch_types=[
            pltpu.VMEM((EW,), jnp.int32),
            pltpu.VMEM((EW,), jnp.float32),
            pltpu.VMEM((NP,), jnp.float32),
            pltpu.VMEM((NP // 128, 128), jnp.int32),
            pltpu.VMEM_SHARED((NP,), jnp.float32),
        ],
    )
    def k(col_h, w_h, z_h, idg_h, out_h, col_v, w_v, dloc, idg_v, dacc):
        cid = lax.axis_index("c")
        sid = lax.axis_index("s")
        wid = sid * NC + cid
        base = wid * EW
        pltpu.sync_copy(z_h.at[pl.ds(0, ept)], dacc.at[pl.ds(sid * ept, ept)])
        pltpu.sync_copy(z_h, dloc)
        pltpu.sync_copy(idg_h, idg_v)
        pltpu.sync_copy(col_h.at[pl.ds(base, EW)], col_v)
        pltpu.sync_copy(w_h.at[pl.ds(base, EW)], w_v)

        def step(i, carry):
            ib = i * LN
            c = col_v[pl.ds(ib, LN)]
            wv = w_v[pl.ds(ib, LN)]
            plsc.addupdate_scatter(dloc, [c], wv)
            return carry

        lax.fori_loop(0, steps, step, 0)
        plsc.subcore_barrier()
        for j in range(NP // 128):
            pltpu.sync_copy(dloc.at[pl.ds(j * 128, 128)],
                            dacc.at[idg_v.at[j]], add=True)
        plsc.subcore_barrier()
        pltpu.sync_copy(dacc.at[pl.ds(sid * ept, ept)],
                        out_h.at[cid, pl.ds(sid * ept, ept)])

    return k(col, w, zeros, idgrid)


def _dis_kernel(degp):
    def body(dp_ref, o_ref):
        deg = dp_ref[0] + dp_ref[1] + 1.0
        o_ref[...] = jnp.where(deg > 0,
                               lax.rsqrt(jnp.maximum(deg, 1e-12)), 0.0)

    return pl.pallas_call(
        body,
        out_shape=jax.ShapeDtypeStruct(degp.shape[1:], jnp.float32),
    )(degp)


def _matmul_scaled(x, wT, dis2d, br=1000):
    n, kdim = x.shape
    m = wT.shape[1]

    def body(x_ref, w_ref, d_ref, o_ref):
        o_ref[...] = d_ref[...] * jnp.dot(x_ref[...], w_ref[...],
                                          preferred_element_type=jnp.float32)

    return pl.pallas_call(
        body,
        grid=(n // br,),
        in_specs=[pl.BlockSpec((br, kdim), lambda i: (i, 0)),
                  pl.BlockSpec((kdim, m), lambda i: (0, 0)),
                  pl.BlockSpec((br, 1), lambda i: (i, 0))],
        out_specs=pl.BlockSpec((br, m), lambda i: (i, 0)),
        out_shape=jax.ShapeDtypeStruct((n, m), jnp.float32),
    )(x, wT, dis2d)


def _prop(table2, row, col2d, w, zeros):
    n2, dq = table2.shape
    n = n2 // NC
    e = row.shape[0]
    ew = e // NS
    steps = ew // GB
    vsteps = ew // LN
    rpt = n // NS
    spt = steps // NS
    del spt
    mesh = plsc.VectorSubcoreMesh(core_axis_name="c", subcore_axis_name="s",
                                  num_cores=NC, num_subcores=NS)

    @functools.partial(
        pl.kernel, mesh=mesh,
        compiler_params=pltpu.CompilerParams(
            needs_layout_passes=False, use_tc_tiling_on_sc=False),
        out_type=jax.ShapeDtypeStruct((NC, n, dq), jnp.float32),
        scratch_types=[
            pltpu.VMEM((ew,), jnp.int32),
            pltpu.VMEM((steps, GB), jnp.int32),
            pltpu.VMEM((ew,), jnp.float32),
            pltpu.VMEM((2, GB, dq), jnp.float32),
            pltpu.VMEM_SHARED((n, dq), jnp.float32),
            pltpu.SemaphoreType.DMA((2,)),
        ],
    )
    def k(tab_h, row_h, col_h, w_h, z_h, part_h,
          radj_v, col_v, w_v, rv, acc, sem):
        cid = lax.axis_index("c")
        sid = lax.axis_index("s")
        base = sid * ew
        roff = cid * n
        zr = zeros.shape[0]
        pltpu.sync_copy(row_h.at[pl.ds(base, ew)], radj_v)
        pltpu.sync_copy(col_h.at[pl.ds(sid * steps, steps)], col_v)
        pltpu.sync_copy(w_h.at[pl.ds(base, ew)], w_v)
        for j in range(rpt // zr):
            pltpu.sync_copy(z_h, acc.at[pl.ds(sid * rpt + j * zr, zr)])

        def adj(i, carry):
            ib = i * LN
            radj_v[pl.ds(ib, LN)] = radj_v[pl.ds(ib, LN)] + roff
            return carry

        lax.fori_loop(0, vsteps, adj, 0)

        for b in range(2):
            pltpu.async_copy(tab_h.at[radj_v.at[pl.ds(b * GB, GB)]],
                             rv.at[b], sem.at[b])
        plsc.subcore_barrier()

        def outer(kk, carry):
            for b in range(2):
                j = 2 * kk + b
                pltpu.make_async_copy(tab_h.at[pl.ds(0, GB)], rv.at[b],
                                      sem.at[b]).wait()
                for r16 in range(GB // LN):
                    wv = w_v[pl.ds(j * GB + r16 * LN, LN)]
                    for lane in range(LN):
                        s = wv[lane]
                        ri = r16 * LN + lane
                        for jj in range(dq // LN):
                            rv[b, ri, pl.ds(jj * LN, LN)] = (
                                rv[b, ri, pl.ds(jj * LN, LN)] * s)
                pltpu.sync_copy(rv.at[b], acc.at[col_v.at[j]], add=True)

                @pl.when(j + 2 < steps)
                def _():
                    pltpu.async_copy(
                        tab_h.at[radj_v.at[pl.ds((j + 2) * GB, GB)]],
                        rv.at[b], sem.at[b])
            return carry

        lax.fori_loop(0, steps // 2, outer, 0)
        plsc.subcore_barrier()
        pltpu.sync_copy(acc.at[pl.ds(sid * rpt, rpt)],
                        part_h.at[cid, pl.ds(sid * rpt, rpt)])

    return k(table2, row, col2d, w, zeros)


def _tc2(part1, dis2d, xw1p, b1row, w2T, br=1000):
    n, d = xw1p.shape
    m = w2T.shape[1]
    ng = part1.shape[0]
    dh = d // ng

    def body(p_ref, d_ref, x_ref, b_ref, w_ref, o_ref):
        dis = d_ref[...]
        agg = jnp.concatenate([p_ref[g] for g in range(ng)], axis=1)
        h = dis * (agg + x_ref[...]) + b_ref[...]
        h = jnp.maximum(h, 0.0)
        o_ref[...] = dis * jnp.dot(h, w_ref[...],
                                   preferred_element_type=jnp.float32)

    return pl.pallas_call(
        body,
        grid=(n // br,),
        in_specs=[pl.BlockSpec((ng, br, dh), lambda i: (0, i, 0)),
                  pl.BlockSpec((br, 1), lambda i: (i, 0)),
                  pl.BlockSpec((br, d), lambda i: (i, 0)),
                  pl.BlockSpec((1, d), lambda i: (0, 0)),
                  pl.BlockSpec((d, m), lambda i: (0, 0))],
        out_specs=pl.BlockSpec((br, m), lambda i: (i, 0)),
        out_shape=jax.ShapeDtypeStruct((n, m), jnp.float32),
    )(part1, dis2d, xw1p, b1row, w2T)


def _tc3(part2, dis2d, h2p, b2row, br=1000):
    n, m = h2p.shape
    ng = part2.shape[0]
    mh = m // ng

    def body(p_ref, d_ref, h_ref, b_ref, o_ref):
        dis = d_ref[...]
        agg = jnp.concatenate([p_ref[g] for g in range(ng)], axis=1)
        logit = dis * (agg + h_ref[...]) + b_ref[...]
        mx = jnp.max(logit, axis=1, keepdims=True)
        lse = jnp.log(jnp.sum(jnp.exp(logit - mx), axis=1, keepdims=True)) + mx
        o_ref[...] = logit - lse

    return pl.pallas_call(
        body,
        grid=(n // br,),
        in_specs=[pl.BlockSpec((ng, br, mh), lambda i: (0, i, 0)),
                  pl.BlockSpec((br, 1), lambda i: (i, 0)),
                  pl.BlockSpec((br, m), lambda i: (i, 0)),
                  pl.BlockSpec((1, m), lambda i: (0, 0))],
        out_specs=pl.BlockSpec((br, m), lambda i: (i, 0)),
        out_shape=jax.ShapeDtypeStruct((n, m), jnp.float32),
    )(part2, dis2d, h2p, b2row)


def kernel(features, edge_index, edge_weight, W1, b1, W2, b2):
    n, _ = features.shape
    e = edge_index.shape[1]
    row = edge_index[0]
    col = edge_index[1]
    col2d = col.reshape(e // GB, GB)

    z10240 = jnp.zeros((10240,), jnp.float32)
    idgrid = jnp.arange(10240, dtype=jnp.int32).reshape(80, 128)
    degp = _deg_kernel(col, edge_weight, z10240, idgrid)
    disg = _dis_kernel(degp.reshape(NC, 80, 128))
    dis_flat = disg.reshape(-1)
    dis2d = dis_flat[:n].reshape(n, 1)

    xw1p = _matmul_scaled(features, W1.T, dis2d)
    d1 = xw1p.shape[1] // NC
    tab1 = jnp.concatenate(
        [xw1p[:, g * d1:(g + 1) * d1] for g in range(NC)], axis=0)
    z1 = jnp.zeros((125, d1), jnp.float32)
    part1 = _prop(tab1, row, col2d, edge_weight, z1)

    h2p = _tc2(part1, dis2d, xw1p, b1.reshape(1, -1), W2.T)
    d2 = h2p.shape[1] // NC
    tab2 = jnp.concatenate(
        [h2p[:, g * d2:(g + 1) * d2] for g in range(NC)], axis=0)
    z2 = jnp.zeros((125, d2), jnp.float32)
    part2 = _prop(tab2, row, col2d, edge_weight, z2)

    return _tc3(part2, dis2d, h2p, b2.reshape(1, -1))


# speedup vs baseline: 24.5046x; 1.1933x over previous
import functools

import jax
import jax.numpy as jnp
from jax import lax
from jax.experimental import pallas as pl
from jax.experimental.pallas import tpu as pltpu
from jax.experimental.pallas import tpu_sc as plsc

NC = 2
NS = 16
NW = NC * NS
LN = 16
GB = 80
NBUF = 5
GA = 3


def _deg_kernel(col, w, zeros, idgrid):
    E = col.shape[0]
    EW = E // NW
    steps = EW // LN
    NP = 10240
    ept = NP // NS
    mesh = plsc.VectorSubcoreMesh(core_axis_name="c", subcore_axis_name="s",
                                  num_cores=NC, num_subcores=NS)

    @functools.partial(
        pl.kernel, mesh=mesh,
        compiler_params=pltpu.CompilerParams(
            needs_layout_passes=False, use_tc_tiling_on_sc=False),
        out_type=jax.ShapeDtypeStruct((NC, NP), jnp.float32),
        scratch_types=[
            pltpu.VMEM((EW,), jnp.int32),
            pltpu.VMEM((EW,), jnp.float32),
            pltpu.VMEM((NP,), jnp.float32),
            pltpu.VMEM((NP // 128, 128), jnp.int32),
            pltpu.VMEM_SHARED((NP,), jnp.float32),
        ],
    )
    def k(col_h, w_h, z_h, idg_h, out_h, col_v, w_v, dloc, idg_v, dacc):
        cid = lax.axis_index("c")
        sid = lax.axis_index("s")
        wid = sid * NC + cid
        base = wid * EW
        pltpu.sync_copy(z_h.at[pl.ds(0, ept)], dacc.at[pl.ds(sid * ept, ept)])
        pltpu.sync_copy(z_h, dloc)
        pltpu.sync_copy(idg_h, idg_v)
        pltpu.sync_copy(col_h.at[pl.ds(base, EW)], col_v)
        pltpu.sync_copy(w_h.at[pl.ds(base, EW)], w_v)

        def step(i, carry):
            ib = i * LN
            c = col_v[pl.ds(ib, LN)]
            wv = w_v[pl.ds(ib, LN)]
            plsc.addupdate_scatter(dloc, [c], wv)
            return carry

        lax.fori_loop(0, steps, step, 0)
        plsc.subcore_barrier()
        for j in range(NP // 128):
            pltpu.sync_copy(dloc.at[pl.ds(j * 128, 128)],
                            dacc.at[idg_v.at[j]], add=True)
        plsc.subcore_barrier()
        pltpu.sync_copy(dacc.at[pl.ds(sid * ept, ept)],
                        out_h.at[cid, pl.ds(sid * ept, ept)])

    return k(col, w, zeros, idgrid)


def _dis_kernel(degp):
    def body(dp_ref, o_ref):
        deg = dp_ref[0] + dp_ref[1] + 1.0
        o_ref[...] = jnp.where(deg > 0,
                               lax.rsqrt(jnp.maximum(deg, 1e-12)), 0.0)

    return pl.pallas_call(
        body,
        out_shape=jax.ShapeDtypeStruct(degp.shape[1:], jnp.float32),
    )(degp)


def _matmul_scaled(x, wT, dis2d, br=1000):
    n, kdim = x.shape
    m = wT.shape[1]
    mh = m // NC

    def body(x_ref, w_ref, d_ref, o_ref):
        res = d_ref[...] * jnp.dot(x_ref[...], w_ref[...],
                                   preferred_element_type=jnp.float32)
        for g in range(NC):
            o_ref[g] = res[:, g * mh:(g + 1) * mh]

    return pl.pallas_call(
        body,
        grid=(n // br,),
        in_specs=[pl.BlockSpec((br, kdim), lambda i: (i, 0)),
                  pl.BlockSpec((kdim, m), lambda i: (0, 0)),
                  pl.BlockSpec((br, 1), lambda i: (i, 0))],
        out_specs=pl.BlockSpec((NC, br, mh), lambda i: (0, i, 0)),
        out_shape=jax.ShapeDtypeStruct((NC, n, mh), jnp.float32),
    )(x, wT, dis2d)


def _prop(table2, row, col2d, w, zeros):
    n2, dq = table2.shape
    n = n2 // NC
    e = row.shape[0]
    ew = e // NS
    steps = ew // GB
    vsteps = ew // LN
    rpt = n // NS
    spt = steps // NS
    del spt
    mesh = plsc.VectorSubcoreMesh(core_axis_name="c", subcore_axis_name="s",
                                  num_cores=NC, num_subcores=NS)

    @functools.partial(
        pl.kernel, mesh=mesh,
        compiler_params=pltpu.CompilerParams(
            needs_layout_passes=False, use_tc_tiling_on_sc=False),
        out_type=jax.ShapeDtypeStruct((NC, n, dq), jnp.float32),
        scratch_types=[
            pltpu.VMEM((ew,), jnp.int32),
            pltpu.VMEM((steps, GB), jnp.int32),
            pltpu.VMEM((ew,), jnp.float32),
            pltpu.VMEM((NBUF, GB, dq), jnp.float32),
            pltpu.VMEM_SHARED((n, dq), jnp.float32),
            pltpu.SemaphoreType.DMA((NBUF,)),
            pltpu.SemaphoreType.DMA((NBUF,)),
        ],
    )
    def k(tab_h, row_h, col_h, w_h, z_h, part_h,
          radj_v, col_v, w_v, rv, acc, semg, sems):
        cid = lax.axis_index("c")
        sid = lax.axis_index("s")
        base = sid * ew
        roff = cid * n
        zr = zeros.shape[0]
        pltpu.sync_copy(row_h.at[pl.ds(base, ew)], radj_v)
        pltpu.sync_copy(col_h.at[pl.ds(sid * steps, steps)], col_v)
        pltpu.sync_copy(w_h.at[pl.ds(base, ew)], w_v)
        for j in range(rpt // zr):
            pltpu.sync_copy(z_h, acc.at[pl.ds(sid * rpt + j * zr, zr)])

        def adj(i, carry):
            ib = i * LN
            radj_v[pl.ds(ib, LN)] = radj_v[pl.ds(ib, LN)] + roff
            return carry

        lax.fori_loop(0, vsteps, adj, 0)

        def wait_gather(b):
            pltpu.make_async_copy(tab_h.at[pl.ds(0, GB)], rv.at[b],
                                  semg.at[b]).wait()

        def wait_scatter(b):
            pltpu.make_async_copy(rv.at[b], acc.at[pl.ds(0, GB)],
                                  sems.at[b]).wait()

        for b in range(GA):
            pltpu.async_copy(tab_h.at[radj_v.at[pl.ds(b * GB, GB)]],
                             rv.at[b], semg.at[b])
        plsc.subcore_barrier()

        def outer(kk, carry):
            for b in range(NBUF):
                j = NBUF * kk + b
                wait_gather(b)
                for r16 in range(GB // LN):
                    wv = w_v[pl.ds(j * GB + r16 * LN, LN)]
                    for lane in range(LN):
                        s = wv[lane]
                        ri = r16 * LN + lane
                        for jj in range(dq // LN):
                            rv[b, ri, pl.ds(jj * LN, LN)] = (
                                rv[b, ri, pl.ds(jj * LN, LN)] * s)
                pltpu.async_copy(rv.at[b], acc.at[col_v.at[j]], sems.at[b],
                                 add=True)
                bn = (b + GA) % NBUF

                @pl.when(jnp.logical_and(j + GA < steps, j >= NBUF - GA))
                def _():
                    wait_scatter(bn)

                @pl.when(j + GA < steps)
                def _():
                    pltpu.async_copy(
                        tab_h.at[radj_v.at[pl.ds((j + GA) * GB, GB)]],
                        rv.at[bn], semg.at[bn])
            return carry

        lax.fori_loop(0, steps // NBUF, outer, 0)
        for b in range(NBUF):
            wait_scatter(b)
        plsc.subcore_barrier()
        pltpu.sync_copy(acc.at[pl.ds(sid * rpt, rpt)],
                        part_h.at[cid, pl.ds(sid * rpt, rpt)])

    return k(table2, row, col2d, w, zeros)


def _tc2(part1, dis2d, xw1p, b1row, w2T, br=1000):
    n = xw1p.shape[1]
    d = xw1p.shape[0] * xw1p.shape[2]
    m = w2T.shape[1]
    ng = part1.shape[0]
    dh = d // ng

    mh = m // NC

    def body(p_ref, d_ref, x_ref, b_ref, w_ref, o_ref):
        dis = d_ref[...]
        agg = jnp.concatenate([p_ref[g] for g in range(ng)], axis=1)
        xs = jnp.concatenate([x_ref[g] for g in range(NC)], axis=1)
        h = dis * (agg + xs) + b_ref[...]
        h = jnp.maximum(h, 0.0)
        res = dis * jnp.dot(h, w_ref[...], preferred_element_type=jnp.float32)
        for g in range(NC):
            o_ref[g] = res[:, g * mh:(g + 1) * mh]

    return pl.pallas_call(
        body,
        grid=(n // br,),
        in_specs=[pl.BlockSpec((ng, br, dh), lambda i: (0, i, 0)),
                  pl.BlockSpec((br, 1), lambda i: (i, 0)),
                  pl.BlockSpec((NC, br, d // NC), lambda i: (0, i, 0)),
                  pl.BlockSpec((1, d), lambda i: (0, 0)),
                  pl.BlockSpec((d, m), lambda i: (0, 0))],
        out_specs=pl.BlockSpec((NC, br, mh), lambda i: (0, i, 0)),
        out_shape=jax.ShapeDtypeStruct((NC, n, mh), jnp.float32),
    )(part1, dis2d, xw1p, b1row, w2T)


def _tc3(part2, dis2d, h2p, b2row, br=1000):
    n = h2p.shape[1]
    m = h2p.shape[0] * h2p.shape[2]
    ng = part2.shape[0]
    mh = m // ng

    def body(p_ref, d_ref, h_ref, b_ref, o_ref):
        dis = d_ref[...]
        agg = jnp.concatenate([p_ref[g] for g in range(ng)], axis=1)
        hs = jnp.concatenate([h_ref[g] for g in range(NC)], axis=1)
        logit = dis * (agg + hs) + b_ref[...]
        mx = jnp.max(logit, axis=1, keepdims=True)
        lse = jnp.log(jnp.sum(jnp.exp(logit - mx), axis=1, keepdims=True)) + mx
        o_ref[...] = logit - lse

    return pl.pallas_call(
        body,
        grid=(n // br,),
        in_specs=[pl.BlockSpec((ng, br, mh), lambda i: (0, i, 0)),
                  pl.BlockSpec((br, 1), lambda i: (i, 0)),
                  pl.BlockSpec((NC, br, m // NC), lambda i: (0, i, 0)),
                  pl.BlockSpec((1, m), lambda i: (0, 0))],
        out_specs=pl.BlockSpec((br, m), lambda i: (i, 0)),
        out_shape=jax.ShapeDtypeStruct((n, m), jnp.float32),
    )(part2, dis2d, h2p, b2row)


def kernel(features, edge_index, edge_weight, W1, b1, W2, b2):
    n, _ = features.shape
    e = edge_index.shape[1]
    row = edge_index[0]
    col = edge_index[1]
    col2d = col.reshape(e // GB, GB)

    z10240 = jnp.zeros((10240,), jnp.float32)
    idgrid = jnp.arange(10240, dtype=jnp.int32).reshape(80, 128)
    degp = _deg_kernel(col, edge_weight, z10240, idgrid)
    disg = _dis_kernel(degp.reshape(NC, 80, 128))
    dis_flat = disg.reshape(-1)
    dis2d = dis_flat[:n].reshape(n, 1)

    xw1p = _matmul_scaled(features, W1.T, dis2d)
    d1 = xw1p.shape[2]
    z1 = jnp.zeros((125, d1), jnp.float32)
    part1 = _prop(xw1p.reshape(NC * n, d1), row, col2d, edge_weight, z1)

    h2p = _tc2(part1, dis2d, xw1p, b1.reshape(1, -1), W2.T)
    d2 = h2p.shape[2]
    z2 = jnp.zeros((125, d2), jnp.float32)
    part2 = _prop(h2p.reshape(NC * n, d2), row, col2d, edge_weight, z2)

    return _tc3(part2, dis2d, h2p, b2.reshape(1, -1))
